# Initial kernel scaffold; baseline (speedup 1.0000x reference)
import jax
import jax.numpy as jnp
from jax.experimental import pallas as pl


def kernel(x, edge_index, hyperedge_attr, lin_W, sheaf_W, conv1_W, conv2_W, lin2_W):
    raise NotImplementedError("write your pallas kernel here")


# speedup vs baseline: 4.9763x; 4.9763x over previous
import functools

import jax
import jax.numpy as jnp
from jax import lax
from jax.experimental import pallas as pl
from jax.experimental.pallas import tpu as pltpu, tpu_sc as plsc

N = 10000
E = 10000
NNZ = 160000
F = 128
D = 4
NCLS = 40

NC, NS = 2, 16
NW = NC * NS
C = 128
NCHUNK = NNZ // C
TPW = (NCHUNK + NW - 1) // NW
TPW_SC = (NCHUNK + NS - 1) // NS
ROWS_A = 624

@functools.cache
def _mesh():
    return plsc.VectorSubcoreMesh(core_axis_name="c", subcore_axis_name="s",
                                  num_cores=NC, num_subcores=NS)


def _mmT_body(x_ref, w_ref, o_ref):
    o_ref[0] = jnp.dot(x_ref[...], w_ref[...],
                       preferred_element_type=jnp.float32)


def _mmT(x, w):
    bn = 1000
    return pl.pallas_call(
        _mmT_body,
        grid=(D, N // bn),
        in_specs=[
            pl.BlockSpec((bn, F), lambda d, i: (i, 0)),
            pl.BlockSpec((F, F), lambda d, i: (0, d)),
        ],
        out_specs=pl.BlockSpec((1, bn, F), lambda d, i: (d, i, 0)),
        out_shape=jax.ShapeDtypeStruct((D, N, F), jnp.float32),
    )(x, w)


def _mm_body(x_ref, w_ref, o_ref):
    o_ref[...] = jnp.dot(x_ref[...], w_ref[...],
                         preferred_element_type=jnp.float32)


def _mm(x, w):
    bn = 2000
    m = x.shape[0]
    return pl.pallas_call(
        _mm_body,
        grid=(m // bn,),
        in_specs=[
            pl.BlockSpec((bn, F), lambda i: (i, 0)),
            pl.BlockSpec((F, F), lambda i: (0, 0)),
        ],
        out_specs=pl.BlockSpec((bn, F), lambda i: (i, 0)),
        out_shape=jax.ShapeDtypeStruct((m, F), jnp.float32),
    )(x, w)


def _meanproj_body(x_ref, w_ref, o_ref):
    xs = (x_ref[0] + x_ref[1] + x_ref[2] + x_ref[3]) * 0.25
    o_ref[...] = jnp.dot(xs, w_ref[...], preferred_element_type=jnp.float32)


def _meanproj(xt, w16):
    bn = 1000
    return pl.pallas_call(
        _meanproj_body,
        grid=(N // bn,),
        in_specs=[
            pl.BlockSpec((D, bn, F), lambda i: (0, i, 0)),
            pl.BlockSpec((F, 16), lambda i: (0, 0)),
        ],
        out_specs=pl.BlockSpec((bn, 16), lambda i: (i, 0)),
        out_shape=jax.ShapeDtypeStruct((N, 16), jnp.float32),
    )(xt, w16)


def _deginv_body(p_ref, o_ref):
    s = p_ref[0] + p_ref[1]
    o_ref[...] = jnp.where(s != 0.0, 1.0 / s, 0.0)


def _deginv(p):
    return pl.pallas_call(
        _deginv_body,
        grid=(1,),
        in_specs=[pl.BlockSpec((2, N, 16), lambda i: (0, 0, 0))],
        out_specs=pl.BlockSpec((N, 16), lambda i: (0, 0)),
        out_shape=jax.ShapeDtypeStruct((N, 16), jnp.float32),
    )(p)


def _scale_body(elu, x_ref, s_ref, o_ref):
    d = pl.program_id(0)
    lane = lax.broadcasted_iota(jnp.int32, s_ref.shape, 1)
    sc = jnp.sum(jnp.where(lane == d, s_ref[...], 0.0), axis=1, keepdims=True)
    v = x_ref[0] * sc
    if elu:
        v = jnp.where(v > 0.0, v, jnp.exp(jnp.minimum(v, 0.0)) - 1.0)
    o_ref[0] = v


def _scale(xt, s, elu):
    bn = 2000
    return pl.pallas_call(
        functools.partial(_scale_body, elu),
        grid=(D, N // bn),
        in_specs=[
            pl.BlockSpec((1, bn, F), lambda d, i: (d, i, 0)),
            pl.BlockSpec((bn, 16), lambda d, i: (i, 0)),
        ],
        out_specs=pl.BlockSpec((1, bn, F), lambda d, i: (d, i, 0)),
        out_shape=jax.ShapeDtypeStruct((D, N, F), jnp.float32),
    )(xt, s)


def _finmm_body(x_ref, w_ref, o_ref):
    @pl.when(pl.program_id(1) == 0)
    def _():
        o_ref[...] = jnp.zeros_like(o_ref)
    o_ref[...] += jnp.dot(x_ref[0], w_ref[0],
                          preferred_element_type=jnp.float32)


def _finmm(ht, w2):
    bn = 1000
    return pl.pallas_call(
        _finmm_body,
        grid=(N // bn, D),
        in_specs=[
            pl.BlockSpec((1, bn, F), lambda i, d: (d, i, 0)),
            pl.BlockSpec((1, F, F), lambda i, d: (d, 0, 0)),
        ],
        out_specs=pl.BlockSpec((bn, F), lambda i, d: (i, 0)),
        out_shape=jax.ShapeDtypeStruct((N, F), jnp.float32),
    )(ht, w2)


def _iota16():
    return lax.iota(jnp.int32, 16)


def _sheaf_body(a_hbm, b_hbm, row_hbm, col_hbm,
                alpha_hbm, degv_hbm, dege_hbm,
                ri_v, ci_v, ga, gb, sbuf, zb, accv, acce, sem):
    c = lax.axis_index("c")
    s = lax.axis_index("s")
    w = s * NC + c

    def _z(i, _):
        zb[i] = jnp.zeros((16,), jnp.float32)
        return 0
    lax.fori_loop(0, ROWS_A, _z, 0)
    for acc in (accv, acce):
        pltpu.sync_copy(zb, acc.at[pl.ds(s * ROWS_A, ROWS_A)])

        @pl.when(s == NS - 1)
        def _():
            pltpu.sync_copy(zb.at[pl.ds(0, 16)],
                            acc.at[pl.ds(NS * ROWS_A, 16)])
    plsc.subcore_barrier()

    def _chunk(t, _):
        cid = w + NW * t

        @pl.when(cid < NCHUNK)
        def _():
            base = cid * C
            pltpu.sync_copy(row_hbm.at[pl.ds(base, C)], ri_v)
            pltpu.sync_copy(col_hbm.at[pl.ds(base, C)], ci_v)
            cp1 = pltpu.async_copy(a_hbm.at[ri_v], ga, sem)
            cp2 = pltpu.async_copy(b_hbm.at[ci_v], gb, sem)
            cp1.wait()
            cp2.wait()

            def _row(r, _):
                v = ga[r] + gb[r]
                sbuf[r] = 1.0 / (1.0 + jnp.exp(-v))
                return 0
            lax.fori_loop(0, C, _row, 0)

            pltpu.sync_copy(sbuf, alpha_hbm.at[pl.ds(base, C)])
            pltpu.sync_copy(sbuf, accv.at[ri_v], add=True)
            pltpu.sync_copy(sbuf, acce.at[ci_v], add=True)
        return 0

    lax.fori_loop(0, TPW, _chunk, 0)
    plsc.subcore_barrier()

    for acc, hbm in ((accv, degv_hbm), (acce, dege_hbm)):
        pltpu.sync_copy(acc.at[pl.ds(s * ROWS_A, ROWS_A)],
                        hbm.at[c, pl.ds(s * ROWS_A, ROWS_A)])

        @pl.when(s == NS - 1)
        def _():
            pltpu.sync_copy(acc.at[pl.ds(NS * ROWS_A, 16)],
                            hbm.at[c, pl.ds(NS * ROWS_A, 16)])


def _sheaf_sc(a16, b16, row, col):
    return pl.kernel(
        _sheaf_body,
        out_type=[
            jax.ShapeDtypeStruct((NNZ, 16), jnp.float32),
            jax.ShapeDtypeStruct((NC, N, 16), jnp.float32),
            jax.ShapeDtypeStruct((NC, N, 16), jnp.float32),
        ],
        mesh=_mesh(),
        compiler_params=pltpu.CompilerParams(use_tc_tiling_on_sc=False),
        scratch_types=[
            pltpu.VMEM((C,), jnp.int32),
            pltpu.VMEM((C,), jnp.int32),
            pltpu.VMEM((C, 16), jnp.float32),
            pltpu.VMEM((C, 16), jnp.float32),
            pltpu.VMEM((C, 16), jnp.float32),
            pltpu.VMEM((ROWS_A, 16), jnp.float32),
            pltpu.VMEM_SHARED((N, 16), jnp.float32),
            pltpu.VMEM_SHARED((N, 16), jnp.float32),
            pltpu.SemaphoreType.DMA,
        ],
    )(a16, b16, row, col)


def _conv_body(tab_hbm, src_hbm, dst_hbm, alpha_hbm, out_hbm,
               si_v, di_v, al_v, gi_v, gbuf, zb, acc, sem):
    c = lax.axis_index("c")
    s = lax.axis_index("s")

    def _z(i, _):
        for k in range(F // 16):
            zb[i, pl.ds(k * 16, 16)] = jnp.zeros((16,), jnp.float32)
        return 0
    lax.fori_loop(0, ROWS_A // 3, _z, 0)

    for dd in range(2):
        d = c * 2 + dd

        for z in range(3):
            pltpu.sync_copy(
                zb, acc.at[pl.ds(s * ROWS_A + z * (ROWS_A // 3),
                                 ROWS_A // 3)])

        @pl.when(s == NS - 1)
        def _():
            pltpu.sync_copy(zb.at[pl.ds(0, 16)],
                            acc.at[pl.ds(NS * ROWS_A, 16)])
        plsc.subcore_barrier()

        def _chunk(t, _):
            cid = s + NS * t

            @pl.when(cid < NCHUNK)
            def _():
                base = cid * C
                pltpu.sync_copy(src_hbm.at[pl.ds(base, C)], si_v)
                pltpu.sync_copy(dst_hbm.at[pl.ds(base, C)], di_v)
                pltpu.sync_copy(alpha_hbm.at[pl.ds(base, C)], al_v)
                off = d * N
                for g in range(C // 16):
                    gi_v[pl.ds(g * 16, 16)] = si_v[pl.ds(g * 16, 16)] + off
                pltpu.async_copy(tab_hbm.at[gi_v], gbuf, sem).wait()

                dlane = jnp.full((16,), d, jnp.int32)

                def _rowm(rr, _):
                    bc = al_v[rr].at[dlane].get(mode="promise_in_bounds")
                    for k in range(F // 16):
                        gbuf[rr, pl.ds(k * 16, 16)] = (
                            gbuf[rr, pl.ds(k * 16, 16)] * bc)
                    return 0
                lax.fori_loop(0, C, _rowm, 0)

                pltpu.sync_copy(gbuf, acc.at[di_v], add=True)
            return 0

        lax.fori_loop(0, TPW_SC, _chunk, 0)
        plsc.subcore_barrier()

        pltpu.sync_copy(acc.at[pl.ds(s * ROWS_A, ROWS_A)],
                        out_hbm.at[d, pl.ds(s * ROWS_A, ROWS_A)])

        @pl.when(s == NS - 1)
        def _():
            pltpu.sync_copy(acc.at[pl.ds(NS * ROWS_A, 16)],
                            out_hbm.at[d, pl.ds(NS * ROWS_A, 16)])
        plsc.subcore_barrier()


def _conv_sc(table, src, dst, alpha_t):
    return pl.kernel(
        _conv_body,
        out_type=jax.ShapeDtypeStruct((D, E, F), jnp.float32),
        mesh=_mesh(),
        compiler_params=pltpu.CompilerParams(use_tc_tiling_on_sc=False),
        scratch_types=[
            pltpu.VMEM((C,), jnp.int32),
            pltpu.VMEM((C,), jnp.int32),
            pltpu.VMEM((C, 16), jnp.float32),
            pltpu.VMEM((C,), jnp.int32),
            pltpu.VMEM((C, F), jnp.float32),
            pltpu.VMEM((ROWS_A // 3, F), jnp.float32),
            pltpu.VMEM_SHARED((E, F), jnp.float32),
            pltpu.SemaphoreType.DMA,
        ],
    )(table, src, dst, alpha_t)


def kernel(x, edge_index, hyperedge_attr, lin_W, sheaf_W, conv1_W, conv2_W,
           lin2_W):
    row, col = edge_index[0], edge_index[1]

    xt = _mmT(x, lin_W)
    et = _mmT(hyperedge_attr, lin_W)

    w1 = jnp.zeros((F, 16), jnp.float32).at[:, :D].set(sheaf_W[:F])
    w2 = jnp.zeros((F, 16), jnp.float32).at[:, :D].set(sheaf_W[F:])
    a16 = _meanproj(xt, w1)
    b16 = _meanproj(et, w2)

    alpha_t, degv_p, dege_p = _sheaf_sc(a16, b16, row, col)
    dv = _deginv(degv_p)
    de = _deginv(dege_p)

    h = xt
    for li, W in ((0, conv1_W), (1, conv2_W)):
        xf = _mm(h.reshape(D * N, F), W)
        m = _conv_sc(xf, row, col, alpha_t)
        m = _scale(m, de, elu=False)
        o = _conv_sc(m.reshape(D * E, F), col, row, alpha_t)
        h = _scale(o, dv, elu=(li == 0))

    w2p = jnp.zeros((D, F, F), jnp.float32).at[:, :, :NCLS].set(
        lin2_W.reshape(D, F, NCLS))
    out = _finmm(h, w2p)
    return out[:, :NCLS]


# speedup vs baseline: 5.3510x; 1.0753x over previous
import functools

import jax
import jax.numpy as jnp
from jax import lax
from jax.experimental import pallas as pl
from jax.experimental.pallas import tpu as pltpu, tpu_sc as plsc

N = 10000
E = 10000
NNZ = 160000
F = 128
D = 4
NCLS = 40

NC, NS = 2, 16
NW = NC * NS
C = 128
NCHUNK = NNZ // C
TPW = (NCHUNK + NW - 1) // NW
NNZ_PAD = 1280 * C
NPCH = NNZ_PAD // C
TPC = NPCH // NS
ROWS_A = 624

@functools.cache
def _mesh():
    return plsc.VectorSubcoreMesh(core_axis_name="c", subcore_axis_name="s",
                                  num_cores=NC, num_subcores=NS)


def _mmT_body(x_ref, w_ref, o_ref):
    o_ref[0] = jnp.dot(x_ref[...], w_ref[...],
                       preferred_element_type=jnp.float32)


def _mmT(x, w):
    bn = 1000
    return pl.pallas_call(
        _mmT_body,
        grid=(D, N // bn),
        in_specs=[
            pl.BlockSpec((bn, F), lambda d, i: (i, 0)),
            pl.BlockSpec((F, F), lambda d, i: (0, d)),
        ],
        out_specs=pl.BlockSpec((1, bn, F), lambda d, i: (d, i, 0)),
        out_shape=jax.ShapeDtypeStruct((D, N, F), jnp.float32),
    )(x, w)


def _mm_body(x_ref, w_ref, o_ref):
    o_ref[...] = jnp.dot(x_ref[...], w_ref[...],
                         preferred_element_type=jnp.float32)


def _mm(x, w):
    bn = 2000
    m = x.shape[0]
    return pl.pallas_call(
        _mm_body,
        grid=(m // bn,),
        in_specs=[
            pl.BlockSpec((bn, F), lambda i: (i, 0)),
            pl.BlockSpec((F, F), lambda i: (0, 0)),
        ],
        out_specs=pl.BlockSpec((bn, F), lambda i: (i, 0)),
        out_shape=jax.ShapeDtypeStruct((m, F), jnp.float32),
    )(x, w)


def _meanproj_body(x_ref, w_ref, o_ref):
    xs = (x_ref[0] + x_ref[1] + x_ref[2] + x_ref[3]) * 0.25
    o_ref[...] = jnp.dot(xs, w_ref[...], preferred_element_type=jnp.float32)


def _meanproj(xt, w16):
    bn = 1000
    return pl.pallas_call(
        _meanproj_body,
        grid=(N // bn,),
        in_specs=[
            pl.BlockSpec((D, bn, F), lambda i: (0, i, 0)),
            pl.BlockSpec((F, 16), lambda i: (0, 0)),
        ],
        out_specs=pl.BlockSpec((bn, 16), lambda i: (i, 0)),
        out_shape=jax.ShapeDtypeStruct((N, 16), jnp.float32),
    )(xt, w16)


def _deginv_body(p_ref, o_ref):
    s = p_ref[0] + p_ref[1]
    o_ref[...] = jnp.where(s != 0.0, 1.0 / s, 0.0)


def _deginv(p):
    return pl.pallas_call(
        _deginv_body,
        grid=(1,),
        in_specs=[pl.BlockSpec((2, N, 16), lambda i: (0, 0, 0))],
        out_specs=pl.BlockSpec((N, 16), lambda i: (0, 0)),
        out_shape=jax.ShapeDtypeStruct((N, 16), jnp.float32),
    )(p)


def _scale_body(elu, x_ref, s_ref, o_ref):
    d = pl.program_id(0)
    lane = lax.broadcasted_iota(jnp.int32, s_ref.shape, 1)
    sc = jnp.sum(jnp.where(lane == d, s_ref[...], 0.0), axis=1, keepdims=True)
    v = x_ref[0] * sc
    if elu:
        v = jnp.where(v > 0.0, v, jnp.exp(jnp.minimum(v, 0.0)) - 1.0)
    o_ref[0] = v


def _scale(xt, s, elu):
    bn = 2000
    return pl.pallas_call(
        functools.partial(_scale_body, elu),
        grid=(D, N // bn),
        in_specs=[
            pl.BlockSpec((1, bn, F), lambda d, i: (d, i, 0)),
            pl.BlockSpec((bn, 16), lambda d, i: (i, 0)),
        ],
        out_specs=pl.BlockSpec((1, bn, F), lambda d, i: (d, i, 0)),
        out_shape=jax.ShapeDtypeStruct((D, N, F), jnp.float32),
    )(xt, s)


def _finmm_body(x_ref, w_ref, o_ref):
    @pl.when(pl.program_id(1) == 0)
    def _():
        o_ref[...] = jnp.zeros_like(o_ref)
    o_ref[...] += jnp.dot(x_ref[0], w_ref[0],
                          preferred_element_type=jnp.float32)


def _finmm(ht, w2):
    bn = 1000
    return pl.pallas_call(
        _finmm_body,
        grid=(N // bn, D),
        in_specs=[
            pl.BlockSpec((1, bn, F), lambda i, d: (d, i, 0)),
            pl.BlockSpec((1, F, F), lambda i, d: (d, 0, 0)),
        ],
        out_specs=pl.BlockSpec((bn, F), lambda i, d: (i, 0)),
        out_shape=jax.ShapeDtypeStruct((N, F), jnp.float32),
    )(ht, w2)


def _iota16():
    return lax.iota(jnp.int32, 16)


def _sheaf_body(a_hbm, b_hbm, row_hbm, col_hbm,
                alpha_hbm, degv_hbm, dege_hbm,
                ri_v, ci_v, ga, gb, sbuf, zb, accv, acce, sem):
    c = lax.axis_index("c")
    s = lax.axis_index("s")
    w = s * NC + c

    def _z(i, _):
        zb[i] = jnp.zeros((16,), jnp.float32)
        return 0
    lax.fori_loop(0, ROWS_A, _z, 0)
    for acc in (accv, acce):
        pltpu.sync_copy(zb, acc.at[pl.ds(s * ROWS_A, ROWS_A)])

        @pl.when(s == NS - 1)
        def _():
            pltpu.sync_copy(zb.at[pl.ds(0, 16)],
                            acc.at[pl.ds(NS * ROWS_A, 16)])
    plsc.subcore_barrier()

    def _chunk(t, _):
        cid = w + NW * t

        @pl.when(cid < NCHUNK)
        def _():
            base = cid * C
            pltpu.sync_copy(row_hbm.at[pl.ds(base, C)], ri_v)
            pltpu.sync_copy(col_hbm.at[pl.ds(base, C)], ci_v)
            cp1 = pltpu.async_copy(a_hbm.at[ri_v], ga, sem)
            cp2 = pltpu.async_copy(b_hbm.at[ci_v], gb, sem)
            cp1.wait()
            cp2.wait()

            def _row(r, _):
                v = ga[r] + gb[r]
                sbuf[r] = 1.0 / (1.0 + jnp.exp(-v))
                return 0
            lax.fori_loop(0, C, _row, 0)

            pltpu.sync_copy(sbuf, alpha_hbm.at[pl.ds(base, C)])
            pltpu.sync_copy(sbuf, accv.at[ri_v], add=True)
            pltpu.sync_copy(sbuf, acce.at[ci_v], add=True)
        return 0

    lax.fori_loop(0, TPW, _chunk, 0)

    @pl.when(jnp.logical_and(c == 0, s < 6))
    def _():
        pltpu.sync_copy(zb, alpha_hbm.at[pl.ds(NNZ + s * ROWS_A, ROWS_A)])

    @pl.when(jnp.logical_and(c == 0, s == 6))
    def _():
        pltpu.sync_copy(zb.at[pl.ds(0, NNZ_PAD - NNZ - 6 * ROWS_A)],
                        alpha_hbm.at[pl.ds(NNZ + 6 * ROWS_A,
                                           NNZ_PAD - NNZ - 6 * ROWS_A)])

    plsc.subcore_barrier()

    for acc, hbm in ((accv, degv_hbm), (acce, dege_hbm)):
        pltpu.sync_copy(acc.at[pl.ds(s * ROWS_A, ROWS_A)],
                        hbm.at[c, pl.ds(s * ROWS_A, ROWS_A)])

        @pl.when(s == NS - 1)
        def _():
            pltpu.sync_copy(acc.at[pl.ds(NS * ROWS_A, 16)],
                            hbm.at[c, pl.ds(NS * ROWS_A, 16)])


def _sheaf_sc(a16, b16, row, col):
    return pl.kernel(
        _sheaf_body,
        out_type=[
            jax.ShapeDtypeStruct((NNZ_PAD, 16), jnp.float32),
            jax.ShapeDtypeStruct((NC, N, 16), jnp.float32),
            jax.ShapeDtypeStruct((NC, N, 16), jnp.float32),
        ],
        mesh=_mesh(),
        compiler_params=pltpu.CompilerParams(use_tc_tiling_on_sc=False),
        scratch_types=[
            pltpu.VMEM((C,), jnp.int32),
            pltpu.VMEM((C,), jnp.int32),
            pltpu.VMEM((C, 16), jnp.float32),
            pltpu.VMEM((C, 16), jnp.float32),
            pltpu.VMEM((C, 16), jnp.float32),
            pltpu.VMEM((ROWS_A, 16), jnp.float32),
            pltpu.VMEM_SHARED((N, 16), jnp.float32),
            pltpu.VMEM_SHARED((N, 16), jnp.float32),
            pltpu.SemaphoreType.DMA,
        ],
    )(a16, b16, row, col)


def _conv_body(tab_hbm, src_hbm, dst_hbm, alpha_hbm, out_hbm,
               si2, di2, al2, gi2, gbuf2, zb, acc,
               isem0, isem1, gsem0, gsem1):
    c = lax.axis_index("c")
    s = lax.axis_index("s")
    isem = (isem0, isem1)
    gsem = (gsem0, gsem1)

    def _z(i, _):
        for k in range(F // 16):
            zb[i, pl.ds(k * 16, 16)] = jnp.zeros((16,), jnp.float32)
        return 0
    lax.fori_loop(0, 16, _z, 0)

    for dd in range(2):
        d = c * 2 + dd
        dlane = jnp.full((16,), d, jnp.int32)

        def _zc(z, _):
            pltpu.sync_copy(zb, acc.at[pl.ds(s * ROWS_A + z * 16, 16)])
            return 0
        lax.fori_loop(0, ROWS_A // 16, _zc, 0)

        @pl.when(s == NS - 1)
        def _():
            pltpu.sync_copy(zb, acc.at[pl.ds(NS * ROWS_A, 16)])
        plsc.subcore_barrier()

        def _issue_idx(slot, t):
            base = (s + NS * t) * C
            pltpu.async_copy(src_hbm.at[pl.ds(base, C)], si2.at[slot],
                             isem[slot])
            pltpu.async_copy(dst_hbm.at[pl.ds(base, C)], di2.at[slot],
                             isem[slot])
            pltpu.async_copy(alpha_hbm.at[pl.ds(base, C)], al2.at[slot],
                             isem[slot])

        def _wait_idx(slot):
            pltpu.make_async_copy(src_hbm.at[pl.ds(0, C)], si2.at[slot],
                                  isem[slot]).wait()
            pltpu.make_async_copy(dst_hbm.at[pl.ds(0, C)], di2.at[slot],
                                  isem[slot]).wait()
            pltpu.make_async_copy(alpha_hbm.at[pl.ds(0, C)], al2.at[slot],
                                  isem[slot]).wait()

        def _issue_gather(slot):
            off = d * N
            for g in range(C // 16):
                gi2[slot, pl.ds(g * 16, 16)] = (
                    si2[slot, pl.ds(g * 16, 16)] + off)
            pltpu.async_copy(tab_hbm.at[gi2.at[slot]], gbuf2.at[slot],
                             gsem[slot])

        def _wait_gather(slot):
            pltpu.make_async_copy(tab_hbm.at[gi2.at[slot]], gbuf2.at[slot],
                                  gsem[slot]).wait()

        def _scale_scatter(slot):
            def _rowm(rr, _):
                bc = al2[slot, rr].at[dlane].get(mode="promise_in_bounds")
                for k in range(F // 16):
                    gbuf2[slot, rr, pl.ds(k * 16, 16)] = (
                        gbuf2[slot, rr, pl.ds(k * 16, 16)] * bc)
                return 0
            lax.fori_loop(0, C, _rowm, 0)
            pltpu.sync_copy(gbuf2.at[slot], acc.at[di2.at[slot]], add=True)

        _issue_idx(0, 0)
        _wait_idx(0)
        _issue_gather(0)
        _issue_idx(1, 1)

        def _piter(t2, _):
            for slot in (0, 1):
                t = 2 * t2 + slot
                other = 1 - slot
                _wait_gather(slot)

                @pl.when(t + 1 < TPC)
                def _():
                    _wait_idx(other)
                    _issue_gather(other)

                _scale_scatter(slot)

                @pl.when(t + 2 < TPC)
                def _():
                    _issue_idx(slot, t + 2)
            return 0

        lax.fori_loop(0, TPC // 2, _piter, 0)
        plsc.subcore_barrier()

        pltpu.sync_copy(acc.at[pl.ds(s * ROWS_A, ROWS_A)],
                        out_hbm.at[d, pl.ds(s * ROWS_A, ROWS_A)])

        @pl.when(s == NS - 1)
        def _():
            pltpu.sync_copy(acc.at[pl.ds(NS * ROWS_A, 16)],
                            out_hbm.at[d, pl.ds(NS * ROWS_A, 16)])
        plsc.subcore_barrier()


def _conv_sc(table, src, dst, alpha_t):
    return pl.kernel(
        _conv_body,
        out_type=jax.ShapeDtypeStruct((D, E, F), jnp.float32),
        mesh=_mesh(),
        compiler_params=pltpu.CompilerParams(use_tc_tiling_on_sc=False),
        scratch_types=[
            pltpu.VMEM((2, C), jnp.int32),
            pltpu.VMEM((2, C), jnp.int32),
            pltpu.VMEM((2, C, 16), jnp.float32),
            pltpu.VMEM((2, C), jnp.int32),
            pltpu.VMEM((2, C, F), jnp.float32),
            pltpu.VMEM((16, F), jnp.float32),
            pltpu.VMEM_SHARED((E, F), jnp.float32),
            pltpu.SemaphoreType.DMA,
            pltpu.SemaphoreType.DMA,
            pltpu.SemaphoreType.DMA,
            pltpu.SemaphoreType.DMA,
        ],
    )(table, src, dst, alpha_t)


def kernel(x, edge_index, hyperedge_attr, lin_W, sheaf_W, conv1_W, conv2_W,
           lin2_W):
    row, col = edge_index[0], edge_index[1]
    zpad = jnp.zeros((NNZ_PAD - NNZ,), jnp.int32)
    row_p = jnp.concatenate([row, zpad])
    col_p = jnp.concatenate([col, zpad])

    xt = _mmT(x, lin_W)
    et = _mmT(hyperedge_attr, lin_W)

    w1 = jnp.zeros((F, 16), jnp.float32).at[:, :D].set(sheaf_W[:F])
    w2 = jnp.zeros((F, 16), jnp.float32).at[:, :D].set(sheaf_W[F:])
    a16 = _meanproj(xt, w1)
    b16 = _meanproj(et, w2)

    alpha_t, degv_p, dege_p = _sheaf_sc(a16, b16, row, col)
    dv = _deginv(degv_p)
    de = _deginv(dege_p)

    h = xt
    for li, W in ((0, conv1_W), (1, conv2_W)):
        xf = _mm(h.reshape(D * N, F), W)
        m = _conv_sc(xf, row_p, col_p, alpha_t)
        m = _scale(m, de, elu=False)
        o = _conv_sc(m.reshape(D * E, F), col_p, row_p, alpha_t)
        h = _scale(o, dv, elu=(li == 0))

    w2p = jnp.zeros((D, F, F), jnp.float32).at[:, :, :NCLS].set(
        lin2_W.reshape(D, F, NCLS))
    out = _finmm(h, w2p)
    return out[:, :NCLS]


# speedup vs baseline: 6.1845x; 1.1558x over previous
import functools

import jax
import jax.numpy as jnp
from jax import lax
from jax.experimental import pallas as pl
from jax.experimental.pallas import tpu as pltpu, tpu_sc as plsc

N = 10000
E = 10000
NNZ = 160000
F = 128
D = 4
NCLS = 40

NC, NS = 2, 16
NW = NC * NS
C = 128
NCHUNK = NNZ // C
TPW = (NCHUNK + NW - 1) // NW
NNZ_PAD = 1280 * C
NPCH = NNZ_PAD // C
TPC = NPCH // NS
ROWS_A = 624

@functools.cache
def _mesh():
    return plsc.VectorSubcoreMesh(core_axis_name="c", subcore_axis_name="s",
                                  num_cores=NC, num_subcores=NS)


def _mmT_body(x_ref, w_ref, o_ref):
    o_ref[0] = jnp.dot(x_ref[...], w_ref[...],
                       preferred_element_type=jnp.float32)


def _mmT(x, w):
    bn = 1000
    return pl.pallas_call(
        _mmT_body,
        grid=(D, N // bn),
        in_specs=[
            pl.BlockSpec((bn, F), lambda d, i: (i, 0)),
            pl.BlockSpec((F, F), lambda d, i: (0, d)),
        ],
        out_specs=pl.BlockSpec((1, bn, F), lambda d, i: (d, i, 0)),
        out_shape=jax.ShapeDtypeStruct((D, N, F), jnp.float32),
    )(x, w)


def _mm_body(x_ref, w_ref, o_ref):
    o_ref[...] = jnp.dot(x_ref[...], w_ref[...],
                         preferred_element_type=jnp.float32)


def _mm(x, w):
    bn = 2000
    m = x.shape[0]
    return pl.pallas_call(
        _mm_body,
        grid=(m // bn,),
        in_specs=[
            pl.BlockSpec((bn, F), lambda i: (i, 0)),
            pl.BlockSpec((F, F), lambda i: (0, 0)),
        ],
        out_specs=pl.BlockSpec((bn, F), lambda i: (i, 0)),
        out_shape=jax.ShapeDtypeStruct((m, F), jnp.float32),
    )(x, w)


def _meanproj_body(x_ref, w_ref, o_ref):
    xs = (x_ref[0] + x_ref[1] + x_ref[2] + x_ref[3]) * 0.25
    o_ref[...] = jnp.dot(xs, w_ref[...], preferred_element_type=jnp.float32)


def _meanproj(xt, w16):
    bn = 1000
    return pl.pallas_call(
        _meanproj_body,
        grid=(N // bn,),
        in_specs=[
            pl.BlockSpec((D, bn, F), lambda i: (0, i, 0)),
            pl.BlockSpec((F, 16), lambda i: (0, 0)),
        ],
        out_specs=pl.BlockSpec((bn, 16), lambda i: (i, 0)),
        out_shape=jax.ShapeDtypeStruct((N, 16), jnp.float32),
    )(xt, w16)


def _deginv_body(p_ref, o_ref):
    s = p_ref[0] + p_ref[1]
    o_ref[...] = jnp.where(s != 0.0, 1.0 / s, 0.0)


def _deginv(p):
    return pl.pallas_call(
        _deginv_body,
        grid=(1,),
        in_specs=[pl.BlockSpec((2, N, 16), lambda i: (0, 0, 0))],
        out_specs=pl.BlockSpec((N, 16), lambda i: (0, 0)),
        out_shape=jax.ShapeDtypeStruct((N, 16), jnp.float32),
    )(p)


def _scale_body(elu, x_ref, s_ref, o_ref):
    d = pl.program_id(0)
    lane = lax.broadcasted_iota(jnp.int32, s_ref.shape, 1)
    sc = jnp.sum(jnp.where(lane == d, s_ref[...], 0.0), axis=1, keepdims=True)
    v = x_ref[0] * sc
    if elu:
        v = jnp.where(v > 0.0, v, jnp.exp(jnp.minimum(v, 0.0)) - 1.0)
    o_ref[0] = v


def _scale(xt, s, elu):
    bn = 2000
    return pl.pallas_call(
        functools.partial(_scale_body, elu),
        grid=(D, N // bn),
        in_specs=[
            pl.BlockSpec((1, bn, F), lambda d, i: (d, i, 0)),
            pl.BlockSpec((bn, 16), lambda d, i: (i, 0)),
        ],
        out_specs=pl.BlockSpec((1, bn, F), lambda d, i: (d, i, 0)),
        out_shape=jax.ShapeDtypeStruct((D, N, F), jnp.float32),
    )(xt, s)


def _finmm_body(x_ref, w_ref, o_ref):
    @pl.when(pl.program_id(1) == 0)
    def _():
        o_ref[...] = jnp.zeros_like(o_ref)
    o_ref[...] += jnp.dot(x_ref[0], w_ref[0],
                          preferred_element_type=jnp.float32)


def _finmm(ht, w2):
    bn = 1000
    return pl.pallas_call(
        _finmm_body,
        grid=(N // bn, D),
        in_specs=[
            pl.BlockSpec((1, bn, F), lambda i, d: (d, i, 0)),
            pl.BlockSpec((1, F, F), lambda i, d: (d, 0, 0)),
        ],
        out_specs=pl.BlockSpec((bn, F), lambda i, d: (i, 0)),
        out_shape=jax.ShapeDtypeStruct((N, F), jnp.float32),
    )(ht, w2)


def _iota16():
    return lax.iota(jnp.int32, 16)


def _sheaf_body(a_hbm, b_hbm, row_hbm, col_hbm,
                alpha_hbm, degv_hbm, dege_hbm,
                ri_v, ci_v, ga, gb, sbuf, zb, accv, acce, sem):
    c = lax.axis_index("c")
    s = lax.axis_index("s")
    w = s * NC + c

    def _z(i, _):
        zb[i] = jnp.zeros((16,), jnp.float32)
        return 0
    lax.fori_loop(0, ROWS_A, _z, 0)
    for acc in (accv, acce):
        pltpu.sync_copy(zb, acc.at[pl.ds(s * ROWS_A, ROWS_A)])

        @pl.when(s == NS - 1)
        def _():
            pltpu.sync_copy(zb.at[pl.ds(0, 16)],
                            acc.at[pl.ds(NS * ROWS_A, 16)])
    plsc.subcore_barrier()

    def _chunk(t, _):
        cid = w + NW * t

        @pl.when(cid < NCHUNK)
        def _():
            base = cid * C
            pltpu.sync_copy(row_hbm.at[pl.ds(base, C)], ri_v)
            pltpu.sync_copy(col_hbm.at[pl.ds(base, C)], ci_v)
            cp1 = pltpu.async_copy(a_hbm.at[ri_v], ga, sem)
            cp2 = pltpu.async_copy(b_hbm.at[ci_v], gb, sem)
            cp1.wait()
            cp2.wait()

            def _row(r, _):
                v = ga[r] + gb[r]
                sbuf[r] = 1.0 / (1.0 + jnp.exp(-v))
                return 0
            lax.fori_loop(0, C, _row, 0)

            pltpu.sync_copy(sbuf, alpha_hbm.at[pl.ds(base, C)])
            pltpu.sync_copy(sbuf, accv.at[ri_v], add=True)
            pltpu.sync_copy(sbuf, acce.at[ci_v], add=True)
        return 0

    lax.fori_loop(0, TPW, _chunk, 0)

    @pl.when(jnp.logical_and(c == 0, s < 6))
    def _():
        pltpu.sync_copy(zb, alpha_hbm.at[pl.ds(NNZ + s * ROWS_A, ROWS_A)])

    @pl.when(jnp.logical_and(c == 0, s == 6))
    def _():
        pltpu.sync_copy(zb.at[pl.ds(0, NNZ_PAD - NNZ - 6 * ROWS_A)],
                        alpha_hbm.at[pl.ds(NNZ + 6 * ROWS_A,
                                           NNZ_PAD - NNZ - 6 * ROWS_A)])

    plsc.subcore_barrier()

    for acc, hbm in ((accv, degv_hbm), (acce, dege_hbm)):
        pltpu.sync_copy(acc.at[pl.ds(s * ROWS_A, ROWS_A)],
                        hbm.at[c, pl.ds(s * ROWS_A, ROWS_A)])

        @pl.when(s == NS - 1)
        def _():
            pltpu.sync_copy(acc.at[pl.ds(NS * ROWS_A, 16)],
                            hbm.at[c, pl.ds(NS * ROWS_A, 16)])


def _sheaf_sc(a16, b16, row, col):
    return pl.kernel(
        _sheaf_body,
        out_type=[
            jax.ShapeDtypeStruct((NNZ_PAD, 16), jnp.float32),
            jax.ShapeDtypeStruct((NC, N, 16), jnp.float32),
            jax.ShapeDtypeStruct((NC, N, 16), jnp.float32),
        ],
        mesh=_mesh(),
        compiler_params=pltpu.CompilerParams(use_tc_tiling_on_sc=False),
        scratch_types=[
            pltpu.VMEM((C,), jnp.int32),
            pltpu.VMEM((C,), jnp.int32),
            pltpu.VMEM((C, 16), jnp.float32),
            pltpu.VMEM((C, 16), jnp.float32),
            pltpu.VMEM((C, 16), jnp.float32),
            pltpu.VMEM((ROWS_A, 16), jnp.float32),
            pltpu.VMEM_SHARED((N, 16), jnp.float32),
            pltpu.VMEM_SHARED((N, 16), jnp.float32),
            pltpu.SemaphoreType.DMA,
        ],
    )(a16, b16, row, col)


def _conv_body(tab_hbm, src_hbm, dst_hbm, alpha_hbm, out_hbm,
               si2, di2, al2, gi2, gbuf2, zb, acc,
               isem0, isem1, isem2, isem3, gsem0, gsem1, ssem0, ssem1):
    c = lax.axis_index("c")
    s = lax.axis_index("s")
    isem = (isem0, isem1, isem2, isem3)
    gsem = (gsem0, gsem1)
    ssem = (ssem0, ssem1)

    def _z(i, _):
        for k in range(F // 16):
            zb[i, pl.ds(k * 16, 16)] = jnp.zeros((16,), jnp.float32)
        return 0
    lax.fori_loop(0, 16, _z, 0)

    for dd in range(2):
        d = c * 2 + dd
        dlane = jnp.full((16,), d, jnp.int32)

        def _zc(z, _):
            pltpu.sync_copy(zb, acc.at[pl.ds(s * ROWS_A + z * 16, 16)])
            return 0
        lax.fori_loop(0, ROWS_A // 16, _zc, 0)

        @pl.when(s == NS - 1)
        def _():
            pltpu.sync_copy(zb, acc.at[pl.ds(NS * ROWS_A, 16)])
        plsc.subcore_barrier()

        def _issue_idx(s4, t):
            base = (s + NS * t) * C
            pltpu.async_copy(src_hbm.at[pl.ds(base, C)], si2.at[s4],
                             isem[s4])
            pltpu.async_copy(dst_hbm.at[pl.ds(base, C)], di2.at[s4],
                             isem[s4])
            pltpu.async_copy(alpha_hbm.at[pl.ds(base, C)], al2.at[s4],
                             isem[s4])

        def _wait_idx(s4):
            pltpu.make_async_copy(src_hbm.at[pl.ds(0, C)], si2.at[s4],
                                  isem[s4]).wait()
            pltpu.make_async_copy(dst_hbm.at[pl.ds(0, C)], di2.at[s4],
                                  isem[s4]).wait()
            pltpu.make_async_copy(alpha_hbm.at[pl.ds(0, C)], al2.at[s4],
                                  isem[s4]).wait()

        def _issue_gather(s2, s4):
            off = d * N
            for g in range(C // 16):
                gi2[s2, pl.ds(g * 16, 16)] = (
                    si2[s4, pl.ds(g * 16, 16)] + off)
            pltpu.async_copy(tab_hbm.at[gi2.at[s2]], gbuf2.at[s2],
                             gsem[s2])

        def _wait_gather(s2):
            pltpu.make_async_copy(tab_hbm.at[gi2.at[s2]], gbuf2.at[s2],
                                  gsem[s2]).wait()

        def _scale(s2, s4):
            def _rowm(q, _):
                for u in range(4):
                    rr = q * 4 + u
                    bc = al2[s4, rr].at[dlane].get(
                        mode="promise_in_bounds")
                    for k in range(F // 16):
                        gbuf2[s2, rr, pl.ds(k * 16, 16)] = (
                            gbuf2[s2, rr, pl.ds(k * 16, 16)] * bc)
                return 0
            lax.fori_loop(0, C // 4, _rowm, 0)

        def _start_scatter(s2, s4):
            return pltpu.async_copy(gbuf2.at[s2], acc.at[di2.at[s4]],
                                    ssem[s2], add=True)

        def _wait_scatter(s2, s4):
            pltpu.make_async_copy(gbuf2.at[s2], acc.at[di2.at[s4]],
                                  ssem[s2]).wait()

        _issue_idx(0, 0)
        _issue_idx(1, 1)
        _issue_idx(2, 2)
        _wait_idx(0)
        _issue_gather(0, 0)

        def _piter(t, _):
            s4 = lax.rem(t, 4)
            for q4 in (0, 1, 2, 3):
                @pl.when(s4 == q4)
                def _():
                    q2 = q4 & 1
                    o2 = 1 - q2
                    n4 = (q4 + 1) % 4
                    p4 = (q4 + 3) % 4
                    _wait_gather(q2)

                    @pl.when(t >= 1)
                    def _():
                        _wait_scatter(o2, p4)

                    @pl.when(t + 1 < TPC)
                    def _():
                        _wait_idx(n4)
                        _issue_gather(o2, n4)

                    _scale(q2, q4)
                    _start_scatter(q2, q4)

                    @pl.when(t + 3 < TPC)
                    def _():
                        _issue_idx(p4, t + 3)
            return 0

        lax.fori_loop(0, TPC, _piter, 0)
        _wait_scatter(1, 3)
        plsc.subcore_barrier()

        pltpu.sync_copy(acc.at[pl.ds(s * ROWS_A, ROWS_A)],
                        out_hbm.at[d, pl.ds(s * ROWS_A, ROWS_A)])

        @pl.when(s == NS - 1)
        def _():
            pltpu.sync_copy(acc.at[pl.ds(NS * ROWS_A, 16)],
                            out_hbm.at[d, pl.ds(NS * ROWS_A, 16)])
        plsc.subcore_barrier()


def _conv_sc(table, src, dst, alpha_t):
    return pl.kernel(
        _conv_body,
        out_type=jax.ShapeDtypeStruct((D, E, F), jnp.float32),
        mesh=_mesh(),
        compiler_params=pltpu.CompilerParams(use_tc_tiling_on_sc=False),
        scratch_types=[
            pltpu.VMEM((4, C), jnp.int32),
            pltpu.VMEM((4, C), jnp.int32),
            pltpu.VMEM((4, C, 16), jnp.float32),
            pltpu.VMEM((2, C), jnp.int32),
            pltpu.VMEM((2, C, F), jnp.float32),
            pltpu.VMEM((16, F), jnp.float32),
            pltpu.VMEM_SHARED((E, F), jnp.float32),
        ] + [pltpu.SemaphoreType.DMA] * 8,
    )(table, src, dst, alpha_t)


def kernel(x, edge_index, hyperedge_attr, lin_W, sheaf_W, conv1_W, conv2_W,
           lin2_W):
    row, col = edge_index[0], edge_index[1]
    zpad = jnp.zeros((NNZ_PAD - NNZ,), jnp.int32)
    row_p = jnp.concatenate([row, zpad])
    col_p = jnp.concatenate([col, zpad])

    xt = _mmT(x, lin_W)
    et = _mmT(hyperedge_attr, lin_W)

    w1 = jnp.zeros((F, 16), jnp.float32).at[:, :D].set(sheaf_W[:F])
    w2 = jnp.zeros((F, 16), jnp.float32).at[:, :D].set(sheaf_W[F:])
    a16 = _meanproj(xt, w1)
    b16 = _meanproj(et, w2)

    alpha_t, degv_p, dege_p = _sheaf_sc(a16, b16, row, col)
    dv = _deginv(degv_p)
    de = _deginv(dege_p)

    h = xt
    for li, W in ((0, conv1_W), (1, conv2_W)):
        xf = _mm(h.reshape(D * N, F), W)
        m = _conv_sc(xf, row_p, col_p, alpha_t)
        m = _scale(m, de, elu=False)
        o = _conv_sc(m.reshape(D * E, F), col_p, row_p, alpha_t)
        h = _scale(o, dv, elu=(li == 0))

    w2p = jnp.zeros((D, F, F), jnp.float32).at[:, :, :NCLS].set(
        lin2_W.reshape(D, F, NCLS))
    out = _finmm(h, w2p)
    return out[:, :NCLS]
